# baseline (speedup 1.0000x reference)
import functools

import jax
import jax.numpy as jnp
from jax import lax
from jax.experimental import pallas as pl
from jax.experimental.pallas import tpu as pltpu
from jax.experimental.pallas import tpu_sc as plsc

N = 10000
D = 128
E = 320000
NC, NS = 2, 16
NW = NC * NS
CH = 128
NCHUNK = 80
EPT = NCHUNK * CH
ERT = E // NW
NPAD = 10112
OPT = NPAD // NS
NFULL = OPT // CH
NREM = OPT - NFULL * CH
WV = 40
NWAVE = NCHUNK // WV

_mesh = plsc.VectorSubcoreMesh(core_axis_name="c", subcore_axis_name="s")


@functools.partial(
    pl.kernel,
    out_type=[
        jax.ShapeDtypeStruct((NC * NPAD, D), jnp.float32),
        jax.ShapeDtypeStruct((NC * NPAD,), jnp.float32),
    ],
    mesh=_mesh,
    scratch_types=[
        pltpu.VMEM((WV, CH), jnp.int32),
        pltpu.VMEM((WV, CH), jnp.int32),
        pltpu.VMEM((CH, D), jnp.float32),
        pltpu.VMEM((CH,), jnp.float32),
        pltpu.VMEM((OPT,), jnp.float32),
        pltpu.VMEM_SHARED((NPAD, D), jnp.float32),
        pltpu.VMEM_SHARED((NPAD,), jnp.float32),
    ],
)
def _phase1(x_hbm, col_hbm, row_hbm, xsum_out, deg_out,
            colv, rowv, rows, onesv, buf1, xsum_sh, deg_sh):
    c = lax.axis_index("c")
    s = lax.axis_index("s")
    wid = c * NS + s

    zero16 = jnp.zeros((16,), jnp.float32)
    one16 = jnp.ones((16,), jnp.float32)

    def _z1(i, _):
        buf1[pl.ds(i * 16, 16)] = zero16
        return 0
    lax.fori_loop(0, OPT // 16, _z1, 0)
    buf1[pl.ds(OPT - 16, 16)] = zero16

    def _o1(i, _):
        onesv[pl.ds(i * 16, 16)] = one16
        return 0
    lax.fori_loop(0, CH // 16, _o1, 0)

    def _zrow(i, _):
        for j in range(D // 16):
            rows[i, pl.ds(j * 16, 16)] = zero16
        return 0
    lax.fori_loop(0, CH, _zrow, 0)

    for k in range(NFULL):
        pltpu.sync_copy(rows, xsum_sh.at[pl.ds(s * OPT + k * CH, CH)])
    pltpu.sync_copy(rows.at[pl.ds(0, NREM)],
                    xsum_sh.at[pl.ds(s * OPT + NFULL * CH, NREM)])
    pltpu.sync_copy(buf1, deg_sh.at[pl.ds(s * OPT, OPT)])
    plsc.subcore_barrier()

    def _wave(w, _):
        pltpu.sync_copy(col_hbm.at[wid, pl.ds(w * WV, WV)], colv)
        pltpu.sync_copy(row_hbm.at[wid, pl.ds(w * WV, WV)], rowv)

        for j in range(WV):
            pltpu.sync_copy(x_hbm.at[colv.at[j]], rows)
            pltpu.sync_copy(rows, xsum_sh.at[rowv.at[j]], add=True)
            pltpu.sync_copy(onesv, deg_sh.at[rowv.at[j]], add=True)
        return 0
    lax.fori_loop(0, NWAVE, _wave, 0)

    plsc.subcore_barrier()

    obase = c * NPAD
    for k in range(NFULL):
        off = s * OPT + k * CH
        pltpu.sync_copy(xsum_sh.at[pl.ds(off, CH)], rows)
        pltpu.sync_copy(rows, xsum_out.at[pl.ds(obase + off, CH)])
    offr = s * OPT + NFULL * CH
    pltpu.sync_copy(xsum_sh.at[pl.ds(offr, NREM)],
                    rows.at[pl.ds(0, NREM)])
    pltpu.sync_copy(rows.at[pl.ds(0, NREM)],
                    xsum_out.at[pl.ds(obase + offr, NREM)])
    pltpu.sync_copy(deg_sh.at[pl.ds(s * OPT, OPT)], buf1)
    pltpu.sync_copy(buf1, deg_out.at[pl.ds(c * NPAD + s * OPT, OPT)])


def _phase2_body(x_ref, xs0_ref, xs1_ref, degp_ref, w_ref,
                 t_ref, s13_ref, recip_ref):
    x = x_ref[...]
    xsum = xs0_ref[...] + xs1_ref[...]
    deg = degp_ref[0, :] + degp_ref[1, :]
    recip = 1.0 / jnp.maximum(deg, 1.0)
    w1 = w_ref[0:1, :]
    w2 = w_ref[1:2, :]
    w3 = w_ref[2:3, :]
    xmean = xsum * recip[:, None]
    t_ref[0, :] = jnp.sum(jnp.abs(x - xmean) * w2, axis=1)
    s13_ref[0, :] = (recip * jnp.sum(xsum * x * w1, axis=1)
                     + jnp.sum(x * w3, axis=1))
    recip_ref[0, :] = recip


_phase2 = pl.pallas_call(
    _phase2_body,
    out_shape=[jax.ShapeDtypeStruct((1, N), jnp.float32)] * 3,
)

CH3 = 128
NCH3 = EPT // CH3


@functools.partial(
    pl.kernel,
    out_type=[jax.ShapeDtypeStruct((NC * NPAD,), jnp.float32)],
    mesh=_mesh,
    scratch_types=[
        pltpu.VMEM((NCH3, CH3), jnp.int32),
        pltpu.VMEM((NCH3, CH3), jnp.int32),
        pltpu.VMEM((CH3,), jnp.float32),
        pltpu.VMEM((OPT,), jnp.float32),
        pltpu.VMEM_SHARED((NPAD,), jnp.float32),
    ],
)
def _phase3(t_hbm, col_hbm, row_hbm, usum_out,
            colv, rowv, tch, buf1, usum_sh):
    c = lax.axis_index("c")
    s = lax.axis_index("s")
    wid = c * NS + s

    zero16 = jnp.zeros((16,), jnp.float32)

    def _z1(i, _):
        buf1[pl.ds(i * 16, 16)] = zero16
        return 0
    lax.fori_loop(0, OPT // 16, _z1, 0)
    buf1[pl.ds(OPT - 16, 16)] = zero16
    pltpu.sync_copy(buf1, usum_sh.at[pl.ds(s * OPT, OPT)])

    pltpu.sync_copy(col_hbm.at[wid], colv)
    pltpu.sync_copy(row_hbm.at[wid], rowv)
    plsc.subcore_barrier()

    for j in range(NCH3):
        pltpu.sync_copy(t_hbm.at[colv.at[j]], tch)
        pltpu.sync_copy(tch, usum_sh.at[rowv.at[j]], add=True)

    plsc.subcore_barrier()
    pltpu.sync_copy(usum_sh.at[pl.ds(s * OPT, OPT)], buf1)
    pltpu.sync_copy(buf1, usum_out.at[pl.ds(c * NPAD + s * OPT, OPT)])


def _phase4_body(s13_ref, up_ref, recip_ref, out_ref):
    u = up_ref[0, :] + up_ref[1, :]
    out_ref[0, :] = jax.nn.sigmoid(s13_ref[0, :] + u * recip_ref[0, :])


_phase4 = pl.pallas_call(
    _phase4_body,
    out_shape=jax.ShapeDtypeStruct((1, N), jnp.float32),
)


def kernel(x, adj, W):
    npad_extra = NPAD - N
    ppt = EPT - ERT
    pad_rows = N + (jnp.arange(NW * ppt, dtype=jnp.int32) % npad_extra)
    row_p = jnp.concatenate(
        [adj[0].reshape(NW, ERT), pad_rows.reshape(NW, ppt)], axis=1)
    col_p = jnp.concatenate(
        [adj[1].reshape(NW, ERT),
         jnp.zeros((NW, ppt), jnp.int32)], axis=1)
    col3 = col_p.reshape(NW, NCHUNK, CH)
    row3 = row_p.reshape(NW, NCHUNK, CH)
    col3b = col_p.reshape(NW, NCH3, CH3)
    row3b = row_p.reshape(NW, NCH3, CH3)
    wr = W.reshape(3, D)

    xsum_p, deg_p = _phase1(x, col3, row3)
    xsp = xsum_p.reshape(NC, NPAD, D)
    degp = deg_p.reshape(NC, NPAD)[:, :N]

    t2, s13, recip = _phase2(x, xsp[0, :N], xsp[1, :N], degp, wr)

    (usum_p,) = _phase3(t2.reshape(N), col3b, row3b)

    score = _phase4(s13, usum_p.reshape(NC, NPAD)[:, :N], recip)
    return score.reshape(N, 1)


# speedup vs baseline: 1.6466x; 1.6466x over previous
import functools

import jax
import jax.numpy as jnp
from jax import lax
from jax.experimental import pallas as pl
from jax.experimental.pallas import tpu as pltpu
from jax.experimental.pallas import tpu_sc as plsc

N = 10000
D = 128
E = 320000
NC, NS = 2, 16
NW = NC * NS
EPT = E // NW
CH = 80
NCHUNK = EPT // CH
NPAD = 10240
OPT = NPAD // NS

_mesh = plsc.VectorSubcoreMesh(core_axis_name="c", subcore_axis_name="s")


@functools.partial(
    pl.kernel,
    out_type=[
        jax.ShapeDtypeStruct((NC, NPAD, D), jnp.float32),
        jax.ShapeDtypeStruct((NC, NPAD), jnp.float32),
    ],
    mesh=_mesh,
    scratch_types=[
        pltpu.VMEM((NCHUNK, CH), jnp.int32),
        pltpu.VMEM((NCHUNK, CH), jnp.int32),
        pltpu.VMEM((CH, D), jnp.float32),
        pltpu.VMEM((CH,), jnp.float32),
        pltpu.VMEM((OPT,), jnp.float32),
        pltpu.VMEM_SHARED((NPAD, D), jnp.float32),
        pltpu.VMEM_SHARED((NPAD,), jnp.float32),
    ],
)
def _phase1(x_hbm, col_hbm, row_hbm, xsum_out, deg_out,
            colv, rowv, rows, onesv, buf1, xsum_sh, deg_sh):
    c = lax.axis_index("c")
    s = lax.axis_index("s")
    wid = c * NS + s

    zero16 = jnp.zeros((16,), jnp.float32)
    one16 = jnp.ones((16,), jnp.float32)

    def _zrow(i, _):
        for j in range(D // 16):
            rows[i, pl.ds(j * 16, 16)] = zero16
        return 0
    lax.fori_loop(0, CH, _zrow, 0)

    def _z1(i, _):
        buf1[pl.ds(i * 16, 16)] = zero16
        return 0
    lax.fori_loop(0, OPT // 16, _z1, 0)

    def _o1(i, _):
        onesv[pl.ds(i * 16, 16)] = one16
        return 0
    lax.fori_loop(0, CH // 16, _o1, 0)

    for k in range(OPT // CH):
        pltpu.sync_copy(rows, xsum_sh.at[pl.ds(s * OPT + k * CH, CH)])
    pltpu.sync_copy(buf1, deg_sh.at[pl.ds(s * OPT, OPT)])

    pltpu.sync_copy(col_hbm.at[wid], colv)
    pltpu.sync_copy(row_hbm.at[wid], rowv)
    plsc.subcore_barrier()

    def _body(i, _):
        pltpu.sync_copy(x_hbm.at[colv.at[i]], rows)
        pltpu.sync_copy(rows, xsum_sh.at[rowv.at[i]], add=True)
        pltpu.sync_copy(onesv, deg_sh.at[rowv.at[i]], add=True)
        return 0
    lax.fori_loop(0, NCHUNK, _body, 0)

    plsc.subcore_barrier()

    for k in range(OPT // CH):
        off = s * OPT + k * CH
        pltpu.sync_copy(xsum_sh.at[pl.ds(off, CH)], rows)
        pltpu.sync_copy(rows, xsum_out.at[c, pl.ds(off, CH)])
    pltpu.sync_copy(deg_sh.at[pl.ds(s * OPT, OPT)], buf1)
    pltpu.sync_copy(buf1, deg_out.at[c, pl.ds(s * OPT, OPT)])


def _phase2_body(x_ref, xs0_ref, xs1_ref, degp_ref, w_ref,
                 t_ref, s13_ref, recip_ref):
    x = x_ref[...]
    xsum = xs0_ref[...] + xs1_ref[...]
    deg = degp_ref[0, :] + degp_ref[1, :]
    recip = 1.0 / jnp.maximum(deg, 1.0)
    w1 = w_ref[0:1, :]
    w2 = w_ref[1:2, :]
    w3 = w_ref[2:3, :]
    xmean = xsum * recip[:, None]
    t_ref[0, :] = jnp.sum(jnp.abs(x - xmean) * w2, axis=1)
    s13_ref[0, :] = (recip * jnp.sum(xsum * x * w1, axis=1)
                     + jnp.sum(x * w3, axis=1))
    recip_ref[0, :] = recip


_phase2 = pl.pallas_call(
    _phase2_body,
    out_shape=[jax.ShapeDtypeStruct((1, N), jnp.float32)] * 3,
)


@functools.partial(
    pl.kernel,
    out_type=[jax.ShapeDtypeStruct((NC, NPAD), jnp.float32)],
    mesh=_mesh,
    scratch_types=[
        pltpu.VMEM((NCHUNK, CH), jnp.int32),
        pltpu.VMEM((NCHUNK, CH), jnp.int32),
        pltpu.VMEM((CH,), jnp.float32),
        pltpu.VMEM((OPT,), jnp.float32),
        pltpu.VMEM_SHARED((NPAD,), jnp.float32),
    ],
)
def _phase3(t_hbm, col_hbm, row_hbm, usum_out,
            colv, rowv, tch, buf1, usum_sh):
    c = lax.axis_index("c")
    s = lax.axis_index("s")
    wid = c * NS + s

    zero16 = jnp.zeros((16,), jnp.float32)

    def _z1(i, _):
        buf1[pl.ds(i * 16, 16)] = zero16
        return 0
    lax.fori_loop(0, OPT // 16, _z1, 0)
    pltpu.sync_copy(buf1, usum_sh.at[pl.ds(s * OPT, OPT)])

    pltpu.sync_copy(col_hbm.at[wid], colv)
    pltpu.sync_copy(row_hbm.at[wid], rowv)
    plsc.subcore_barrier()

    def _body(i, _):
        pltpu.sync_copy(t_hbm.at[colv.at[i]], tch)
        pltpu.sync_copy(tch, usum_sh.at[rowv.at[i]], add=True)
        return 0
    lax.fori_loop(0, NCHUNK, _body, 0)

    plsc.subcore_barrier()
    pltpu.sync_copy(usum_sh.at[pl.ds(s * OPT, OPT)], buf1)
    pltpu.sync_copy(buf1, usum_out.at[c, pl.ds(s * OPT, OPT)])


def _phase4_body(s13_ref, up_ref, recip_ref, out_ref):
    u = up_ref[0, :] + up_ref[1, :]
    out_ref[0, :] = jax.nn.sigmoid(s13_ref[0, :] + u * recip_ref[0, :])


_phase4 = pl.pallas_call(
    _phase4_body,
    out_shape=jax.ShapeDtypeStruct((1, N), jnp.float32),
)


def kernel(x, adj, W):
    row = adj[0]
    col = adj[1]
    col3 = col.reshape(NW, NCHUNK, CH)
    row3 = row.reshape(NW, NCHUNK, CH)
    wr = W.reshape(3, D)

    xsum_p, deg_p = _phase1(x, col3, row3)
    xs0 = xsum_p[0, :N]
    xs1 = xsum_p[1, :N]
    degp = deg_p[:, :N]

    t2, s13, recip = _phase2(x, xs0, xs1, degp, wr)

    (usum_p,) = _phase3(t2.reshape(N), col3, row3)

    score = _phase4(s13, usum_p[:, :N], recip)
    return score.reshape(N, 1)


# speedup vs baseline: 2.1339x; 1.2960x over previous
import functools

import jax
import jax.numpy as jnp
from jax import lax
from jax.experimental import pallas as pl
from jax.experimental.pallas import tpu as pltpu
from jax.experimental.pallas import tpu_sc as plsc

N = 10000
D = 128
E = 320000
NC, NS = 2, 16
NW = NC * NS
EPT = E // NW
CH = 80
NCHUNK = EPT // CH
NPAD = 10240
OPT = NPAD // NS

_mesh = plsc.VectorSubcoreMesh(core_axis_name="c", subcore_axis_name="s")


@functools.partial(
    pl.kernel,
    out_type=[
        jax.ShapeDtypeStruct((NC, NPAD, D), jnp.float32),
        jax.ShapeDtypeStruct((NC, NPAD), jnp.float32),
    ],
    mesh=_mesh,
    scratch_types=[
        pltpu.VMEM((NCHUNK, CH), jnp.int32),
        pltpu.VMEM((NCHUNK, CH), jnp.int32),
        pltpu.VMEM((CH, D), jnp.float32),
        pltpu.VMEM((CH,), jnp.float32),
        pltpu.VMEM((OPT,), jnp.float32),
        pltpu.VMEM_SHARED((NPAD, D), jnp.float32),
        pltpu.VMEM_SHARED((NPAD,), jnp.float32),
    ],
)
def _phase1(x_hbm, col_hbm, row_hbm, xsum_out, deg_out,
            colv, rowv, rows, onesv, buf1, xsum_sh, deg_sh):
    c = lax.axis_index("c")
    s = lax.axis_index("s")
    wid = c * NS + s

    zero16 = jnp.zeros((16,), jnp.float32)
    one16 = jnp.ones((16,), jnp.float32)

    def _zrow(i, _):
        for j in range(D // 16):
            rows[i, pl.ds(j * 16, 16)] = zero16
        return 0
    lax.fori_loop(0, CH, _zrow, 0)

    def _z1(i, _):
        buf1[pl.ds(i * 16, 16)] = zero16
        return 0
    lax.fori_loop(0, OPT // 16, _z1, 0)

    def _o1(i, _):
        onesv[pl.ds(i * 16, 16)] = one16
        return 0
    lax.fori_loop(0, CH // 16, _o1, 0)

    for k in range(OPT // CH):
        pltpu.sync_copy(rows, xsum_sh.at[pl.ds(s * OPT + k * CH, CH)])
    pltpu.sync_copy(buf1, deg_sh.at[pl.ds(s * OPT, OPT)])

    pltpu.sync_copy(col_hbm.at[wid], colv)
    pltpu.sync_copy(row_hbm.at[wid], rowv)
    plsc.subcore_barrier()

    def _body(i, _):
        pltpu.sync_copy(x_hbm.at[colv.at[i]], rows)
        pltpu.sync_copy(rows, xsum_sh.at[rowv.at[i]], add=True)
        pltpu.sync_copy(onesv, deg_sh.at[rowv.at[i]], add=True)
        return 0
    lax.fori_loop(0, NCHUNK, _body, 0)

    plsc.subcore_barrier()

    for k in range(OPT // CH):
        off = s * OPT + k * CH
        pltpu.sync_copy(xsum_sh.at[pl.ds(off, CH)], rows)
        pltpu.sync_copy(rows, xsum_out.at[c, pl.ds(off, CH)])
    pltpu.sync_copy(deg_sh.at[pl.ds(s * OPT, OPT)], buf1)
    pltpu.sync_copy(buf1, deg_out.at[c, pl.ds(s * OPT, OPT)])


def _phase2_body(x_ref, xs0_ref, xs1_ref, degp_ref, w_ref,
                 t_ref, s13_ref, recip_ref):
    x = x_ref[...]
    xsum = xs0_ref[...] + xs1_ref[...]
    deg = degp_ref[0, :] + degp_ref[1, :]
    recip = 1.0 / jnp.maximum(deg, 1.0)
    w1 = w_ref[0:1, :]
    w2 = w_ref[1:2, :]
    w3 = w_ref[2:3, :]
    xmean = xsum * recip[:, None]
    t_ref[0, :] = jnp.sum(jnp.abs(x - xmean) * w2, axis=1)
    s13_ref[0, :] = (recip * jnp.sum(xsum * x * w1, axis=1)
                     + jnp.sum(x * w3, axis=1))
    recip_ref[0, :] = recip


_phase2 = pl.pallas_call(
    _phase2_body,
    out_shape=[jax.ShapeDtypeStruct((1, N), jnp.float32)] * 3,
)


@functools.partial(
    pl.kernel,
    out_type=[jax.ShapeDtypeStruct((NW * NPAD,), jnp.float32)],
    mesh=_mesh,
    scratch_types=[
        pltpu.VMEM((NCHUNK, CH), jnp.int32),
        pltpu.VMEM((NCHUNK, CH), jnp.int32),
        pltpu.VMEM((N,), jnp.float32),
        pltpu.VMEM((NPAD,), jnp.float32),
    ],
    compiler_params=pltpu.CompilerParams(needs_layout_passes=False),
)
def _phase3(t_hbm, col_hbm, row_hbm, usum_out, colv, rowv, tv, usumv):
    c = lax.axis_index("c")
    s = lax.axis_index("s")
    wid = c * NS + s

    zero16 = jnp.zeros((16,), jnp.float32)

    def _z1(i, _):
        usumv[pl.ds(i * 16, 16)] = zero16
        return 0
    lax.fori_loop(0, NPAD // 16, _z1, 0)

    pltpu.sync_copy(col_hbm.at[wid], colv)
    pltpu.sync_copy(row_hbm.at[wid], rowv)
    pltpu.sync_copy(t_hbm, tv)

    def _body(i, _):
        for j in range(CH // 16):
            c16 = colv[i, pl.ds(j * 16, 16)]
            r16 = rowv[i, pl.ds(j * 16, 16)]
            t16 = plsc.load_gather(tv, [c16])
            plsc.addupdate_scatter(usumv, [r16], t16)
        return 0
    lax.fori_loop(0, NCHUNK, _body, 0)

    pltpu.sync_copy(usumv, usum_out.at[pl.ds(wid * NPAD, NPAD)])


def _phase4_body(s13_ref, up_ref, recip_ref, out_ref):
    u = jnp.sum(up_ref[...], axis=0)
    out_ref[0, :] = jax.nn.sigmoid(s13_ref[0, :] + u * recip_ref[0, :])


_phase4 = pl.pallas_call(
    _phase4_body,
    out_shape=jax.ShapeDtypeStruct((1, N), jnp.float32),
)


def kernel(x, adj, W):
    row = adj[0]
    col = adj[1]
    col3 = col.reshape(NW, NCHUNK, CH)
    row3 = row.reshape(NW, NCHUNK, CH)
    wr = W.reshape(3, D)

    xsum_p, deg_p = _phase1(x, col3, row3)
    xs0 = xsum_p[0, :N]
    xs1 = xsum_p[1, :N]
    degp = deg_p[:, :N]

    t2, s13, recip = _phase2(x, xs0, xs1, degp, wr)

    (usum_p,) = _phase3(t2.reshape(N), col3, row3)

    score = _phase4(s13, usum_p.reshape(NW, NPAD)[:, :N], recip)
    return score.reshape(N, 1)
